# Initial kernel scaffold; baseline (speedup 1.0000x reference)
import jax
import jax.numpy as jnp
from jax.experimental import pallas as pl


def kernel(x, tables, W_att, b_att, W_dense, b_dense, W_final, b_final):
    raise NotImplementedError("write your pallas kernel here")


# speedup vs baseline: 1.3487x; 1.3487x over previous
import functools

import jax
import jax.numpy as jnp
from jax import lax
from jax.experimental import pallas as pl
from jax.experimental.pallas import tpu as pltpu
from jax.experimental.pallas import tpu_sc as plsc

_B = 4096
_ND = 13
_NS = 26
_VOCAB = 100000
_EMB = 16
_ATT = 8
_LANE = 128

_NW = 32
_ROWS = _B * _NS
_RPW = _ROWS // _NW
_CH = 128
_NCH = _RPW // _CH


def _sc_gather(tab, idx3):
    @functools.partial(
        pl.kernel,
        mesh=plsc.VectorSubcoreMesh(core_axis_name="c", subcore_axis_name="s"),
        compiler_params=pltpu.CompilerParams(use_tc_tiling_on_sc=False),
        out_type=jax.ShapeDtypeStruct((_NW, _RPW * _EMB // 128, 128), jnp.float32),
    scratch_types=[
            pltpu.VMEM((_RPW,), jnp.int32),
            pltpu.VMEM((_RPW, _EMB), jnp.float32),
            pltpu.VMEM((_RPW * _EMB // 128, 128), jnp.float32),
            pltpu.SemaphoreType.DMA,
        ],
    )
    def body(tab_hbm, idx_hbm, out_hbm, idx_v, rows_v, out_v, sem):
        wid = lax.axis_index("s") * 2 + lax.axis_index("c")
        pltpu.sync_copy(idx_hbm.at[wid], idx_v)
        copies = [
            pltpu.async_copy(
                tab_hbm.at[idx_v.at[pl.ds(c * _CH, _CH)]],
                rows_v.at[pl.ds(c * _CH, _CH)],
                sem,
            )
            for c in range(_NCH)
        ]
        for cp in copies:
            cp.wait()

        def repack(g, carry):
            for j in range(8):
                out_v[g, pl.ds(j * _EMB, _EMB)] = rows_v[g * 8 + j, :]
            return carry

        lax.fori_loop(0, _RPW // 8, repack, 0)
        pltpu.sync_copy(out_v, out_hbm.at[wid])

    return body(tab, idx3)


def _bf16_round(x):
    u = jax.lax.bitcast_convert_type(x, jnp.uint32)
    u = (u + jnp.uint32(0x7FFF) + ((u >> jnp.uint32(16)) & jnp.uint32(1)))
    u = u & jnp.uint32(0xFFFF0000)
    return jax.lax.bitcast_convert_type(u, jnp.float32)


def _att_body(et_ref, dt_ref, wab_ref, bab_ref, wdb_ref, wfe_ref, wfd_ref,
              bfb_ref, mask_ref, o_ref):
    dt = dt_ref[...].T
    et = et_ref[...]
    V = et[:, None, :, :] * et[None, :, :, :]
    s = mask_ref[...]
    for h in range(_ATT):
        ah = jnp.sum(V * wab_ref[h][None, None], axis=2)
        ah = jnp.maximum(ah + bab_ref[h][None, None], 0.0)
        s = s + ah * wdb_ref[h][None, None]
    m = jnp.max(s, axis=(0, 1))
    ex = jnp.exp(s - m[None, None, :])
    z = jnp.sum(ex, axis=(0, 1))
    w = ex / z[None, None, :]
    att = jnp.sum(w[:, :, None, :] * V, axis=(0, 1))
    attb = _bf16_round(att)
    dtb = _bf16_round(dt)
    logit = (jnp.sum(attb * _bf16_round(wfe_ref[...]), axis=0)
             + jnp.sum(dtb * _bf16_round(wfd_ref[...]), axis=0)
             + bfb_ref[0])
    o_ref[...] = jax.nn.sigmoid(logit)[None, None, :]


def _attention(et, dt, wab, bab, wdb, wfe, wfd, bfb, mask):
    grid = (_B // _LANE,)
    out = pl.pallas_call(
        _att_body,
        grid=grid,
        in_specs=[
            pl.BlockSpec((_NS, _EMB, _LANE), lambda i: (0, 0, i)),
            pl.BlockSpec((_LANE, _ND), lambda i: (i, 0)),
            pl.BlockSpec((_ATT, _EMB, _LANE), lambda i: (0, 0, 0)),
            pl.BlockSpec((_ATT, _LANE), lambda i: (0, 0)),
            pl.BlockSpec((_ATT, _LANE), lambda i: (0, 0)),
            pl.BlockSpec((_EMB, _LANE), lambda i: (0, 0)),
            pl.BlockSpec((_ND, _LANE), lambda i: (0, 0)),
            pl.BlockSpec((1, _LANE), lambda i: (0, 0)),
            pl.BlockSpec((_NS, _NS, _LANE), lambda i: (0, 0, 0)),
        ],
        out_specs=pl.BlockSpec((1, 1, _LANE), lambda i: (i, 0, 0)),
        out_shape=jax.ShapeDtypeStruct((_B // _LANE, 1, _LANE), jnp.float32),
    )(et, dt, wab, bab, wdb, wfe, wfd, bfb, mask)
    return out.reshape(_B)


def kernel(x, tables, W_att, b_att, W_dense, b_dense, W_final, b_final):
    del b_dense
    dt = x[:, :_ND]
    sparse = x[:, _ND:].astype(jnp.int32)
    offs = (jnp.arange(_NS, dtype=jnp.int32) * _VOCAB)[None, :]
    idx3 = (sparse + offs).reshape(_NW, _RPW)
    tab = tables.reshape(_NS * _VOCAB, _EMB)
    gathered = _sc_gather(tab, idx3)
    et = gathered.reshape(_B, _NS * _EMB).T.reshape(_NS, _EMB, _B)

    wab = jnp.broadcast_to(W_att.T[:, :, None], (_ATT, _EMB, _LANE))
    bab = jnp.broadcast_to(b_att[:, None], (_ATT, _LANE))
    wdb = jnp.broadcast_to(W_dense[:, 0][:, None], (_ATT, _LANE))
    wfe = jnp.broadcast_to(W_final[:_EMB, 0][:, None], (_EMB, _LANE))
    wfd = jnp.broadcast_to(W_final[_EMB:, 0][:, None], (_ND, _LANE))
    bfb = jnp.broadcast_to(b_final[:, None], (1, _LANE))
    mask = jnp.where(jnp.eye(_NS, dtype=bool)[:, :, None], -1e30, 0.0)
    mask = jnp.broadcast_to(mask.astype(jnp.float32), (_NS, _NS, _LANE))

    return _attention(et, dt, wab, bab, wdb, wfe, wfd, bfb, mask)
